# Initial kernel scaffold; baseline (speedup 1.0000x reference)
import jax
import jax.numpy as jnp
from jax.experimental import pallas as pl


def kernel(src_org_edge_feat, src_edge_to_time, src_center_node_idx, src_neigh_edge, src_node_features, current_time, label, params):
    raise NotImplementedError("write your pallas kernel here")


# speedup vs baseline: 24.8254x; 24.8254x over previous
import functools

import jax
import jax.numpy as jnp
import numpy as np
from jax import lax
from jax.experimental import pallas as pl
from jax.experimental.pallas import tpu as pltpu
from jax.experimental.pallas import tpu_sc as plsc

N_NODES = 10000
N_EDGES = 320000
NODE_DIM = 128
EDGE_DIM = 16
D = 128
H = 2
DH = D // H
N_LAYERS = 2
B = 1024

_NC, _NS = 2, 16
_NW = _NC * _NS
_NPAD = 10112


def _bdot(a, b):
    return jnp.dot(a.astype(jnp.bfloat16), b.astype(jnp.bfloat16),
                   preferred_element_type=jnp.float32)


def _sc_mesh():
    return plsc.VectorSubcoreMesh(
        core_axis_name="c", subcore_axis_name="s", num_cores=_NC, num_subcores=_NS
    )


def _eht_body(ef_ref, temb_ref, ew1_ref, eb1_ref, ew2_ref, eb2_ref, out_ref):
    r = jax.nn.relu(_bdot(ef_ref[...], ew1_ref[...]) + eb1_ref[...])
    out_ref[...] = _bdot(r, ew2_ref[...]) + eb2_ref[...] + temb_ref[...]


def _edge_ht(ef, t, p, E, BE):
    grid = (E // BE,)
    full = lambda shape: pl.BlockSpec(shape, lambda i: (0, 0))
    temb = jnp.cos(t[:, None] * p["time_w"][None, :] + p["time_b"][None, :])
    return pl.pallas_call(
        _eht_body,
        grid=grid,
        in_specs=[
            pl.BlockSpec((BE, EDGE_DIM), lambda i: (i, 0)),
            pl.BlockSpec((BE, D), lambda i: (i, 0)),
            full((EDGE_DIM, D)),
            full((1, D)),
            full((D, D)),
            full((1, D)),
        ],
        out_specs=pl.BlockSpec((BE, D), lambda i: (i, 0)),
        out_shape=jax.ShapeDtypeStruct((E, D), jnp.float32),
    )(ef, temb, p["edge_w1"], p["edge_b1"][None, :], p["edge_w2"],
      p["edge_b2"][None, :])


def _sc_gather(hq, h, src, dst, E):
    CB = 80
    ec = E // _NW
    nblk = ec // CB

    @functools.partial(
        pl.kernel,
        out_type=(
            jax.ShapeDtypeStruct((E, D), jnp.float32),
            jax.ShapeDtypeStruct((E, D), jnp.float32),
        ),
        mesh=_sc_mesh(),
        scratch_types=[
            pltpu.VMEM((CB,), jnp.int32),
            pltpu.VMEM((CB,), jnp.int32),
            pltpu.VMEM((CB, D), jnp.float32),
            pltpu.VMEM((CB, D), jnp.float32),
            pltpu.SemaphoreType.DMA,
            pltpu.SemaphoreType.DMA,
        ],
    )
    def k(hq_hbm, h_hbm, src_hbm, dst_hbm, oq_hbm, os_hbm,
          dbuf, sbuf, qbuf, hsbuf, sem1, sem2):
        wid = lax.axis_index("s") * _NC + lax.axis_index("c")

        def body(i, carry):
            base = wid * ec + i * CB
            pltpu.sync_copy(dst_hbm.at[pl.ds(base, CB)], dbuf)
            pltpu.sync_copy(src_hbm.at[pl.ds(base, CB)], sbuf)
            c1 = pltpu.async_copy(hq_hbm.at[dbuf], qbuf, sem1)
            c2 = pltpu.async_copy(h_hbm.at[sbuf], hsbuf, sem2)
            c1.wait()
            c2.wait()
            pltpu.sync_copy(qbuf, oq_hbm.at[pl.ds(base, CB)])
            pltpu.sync_copy(hsbuf, os_hbm.at[pl.ds(base, CB)])
            return carry

        lax.fori_loop(0, nblk, body, 0)

    return k(hq, h, src, dst)


def _payload_body(gq_ref, gs_ref, eht_ref, wk_ref, wv_ref, pay0_ref, pay1_ref):
    gq = gq_ref[...]
    msg = gs_ref[...] + eht_ref[...]
    kk = _bdot(msg, wk_ref[...])
    v = _bdot(msg, wv_ref[...])
    s0 = jnp.sum(gq[:, :DH] * kk[:, :DH], axis=1, keepdims=True) * (1.0 / np.sqrt(DH))
    s1 = jnp.sum(gq[:, DH:] * kk[:, DH:], axis=1, keepdims=True) * (1.0 / np.sqrt(DH))
    e0 = jnp.exp(s0)
    e1 = jnp.exp(s1)
    n = gq.shape[0]
    z = jnp.zeros((n, DH - 1), jnp.float32)
    pay0_ref[...] = jnp.concatenate([e0 * v[:, :DH], e0, z], axis=1)
    pay1_ref[...] = jnp.concatenate([e1 * v[:, DH:], e1, z], axis=1)


def _edge_payload(gq, gs, eht, wk, wv, E, BE):
    grid = (E // BE,)
    full = lambda shape: pl.BlockSpec(shape, lambda i: (0, 0))
    return pl.pallas_call(
        _payload_body,
        grid=grid,
        in_specs=[
            pl.BlockSpec((BE, D), lambda i: (i, 0)),
            pl.BlockSpec((BE, D), lambda i: (i, 0)),
            pl.BlockSpec((BE, D), lambda i: (i, 0)),
            full((D, D)),
            full((D, D)),
        ],
        out_specs=[
            pl.BlockSpec((BE, D), lambda i: (i, 0)),
            pl.BlockSpec((BE, D), lambda i: (i, 0)),
        ],
        out_shape=[
            jax.ShapeDtypeStruct((E, D), jnp.float32),
            jax.ShapeDtypeStruct((E, D), jnp.float32),
        ],
    )(gq, gs, eht, wk, wv)


def _sc_scatter(pay0, pay1, dst, zeros, E):
    CB = 80
    ec = E // _NW
    nblk = ec // CB
    rpt = _NPAD // _NS

    @functools.partial(
        pl.kernel,
        out_type=jax.ShapeDtypeStruct((_NC, H, _NPAD, D), jnp.float32),
        mesh=_sc_mesh(),
        scratch_types=[
            pltpu.VMEM((CB,), jnp.int32),
            pltpu.VMEM((CB, D), jnp.float32),
            pltpu.VMEM_SHARED((_NPAD, D), jnp.float32),
        ],
    )
    def k(pay0_hbm, pay1_hbm, dst_hbm, zeros_hbm, out_hbm, dbuf, pbuf, acc):
        cid = lax.axis_index("c")
        sid = lax.axis_index("s")
        wid = sid * _NC + cid
        r0 = sid * rpt

        def run_phase(pay_hbm, hout):
            def body(i, carry):
                base = wid * ec + i * CB
                pltpu.sync_copy(dst_hbm.at[pl.ds(base, CB)], dbuf)
                pltpu.sync_copy(pay_hbm.at[pl.ds(base, CB)], pbuf)
                pltpu.sync_copy(pbuf, acc.at[dbuf], add=True)
                return carry

            lax.fori_loop(0, nblk, body, 0)
            plsc.subcore_barrier()
            pltpu.sync_copy(acc.at[pl.ds(r0, rpt)], out_hbm.at[cid, hout, pl.ds(r0, rpt)])

        pltpu.sync_copy(zeros_hbm.at[pl.ds(r0, rpt)], acc.at[pl.ds(r0, rpt)])
        plsc.subcore_barrier()
        run_phase(pay0_hbm, 0)
        pltpu.sync_copy(zeros_hbm.at[pl.ds(r0, rpt)], acc.at[pl.ds(r0, rpt)])
        plsc.subcore_barrier()
        run_phase(pay1_hbm, 1)

    return k(pay0, pay1, dst, zeros)


def _agg_from_partials(pp):
    n = pp.shape[2]
    num0 = pp[0, 0, :, :DH] + pp[1, 0, :, :DH]
    num1 = pp[0, 1, :, :DH] + pp[1, 1, :, :DH]
    den0 = pp[0, 0, :, DH:DH + 1] + pp[1, 0, :, DH:DH + 1] + 1e-16
    den1 = pp[0, 1, :, DH:DH + 1] + pp[1, 1, :, DH:DH + 1] + 1e-16
    return jnp.concatenate(
        [num0 / jnp.broadcast_to(den0, (n, DH)),
         num1 / jnp.broadcast_to(den1, (n, DH))], axis=1)


def _combine_body(p_ref, h_ref, wo_ref, wskip_ref, bo_ref, wq_ref,
                  hnew_ref, hq_ref):
    agg = _agg_from_partials(p_ref[...])
    hnew = jax.nn.relu(_bdot(agg, wo_ref[...]) + _bdot(h_ref[...], wskip_ref[...])
                       + bo_ref[...])
    hnew_ref[...] = hnew
    hq_ref[...] = _bdot(hnew, wq_ref[...])


def _combine_final_body(p_ref, h_ref, wo_ref, wskip_ref, bo_ref, hnew_ref):
    agg = _agg_from_partials(p_ref[...])
    hnew_ref[...] = jax.nn.relu(
        _bdot(agg, wo_ref[...]) + _bdot(h_ref[...], wskip_ref[...]) + bo_ref[...])


def _combine(partials, h, wo, wskip, bo, wq_next, BN=2000):
    grid = (N_NODES // BN,)
    full = lambda shape: pl.BlockSpec(shape, lambda i: tuple(0 for _ in shape))
    if wq_next is not None:
        return pl.pallas_call(
            _combine_body,
            grid=grid,
            in_specs=[
                pl.BlockSpec((2, H, BN, D), lambda i: (0, 0, i, 0)),
                pl.BlockSpec((BN, D), lambda i: (i, 0)),
                full((D, D)),
                full((D, D)),
                full((1, D)),
                full((D, D)),
            ],
            out_specs=[
                pl.BlockSpec((BN, D), lambda i: (i, 0)),
                pl.BlockSpec((BN, D), lambda i: (i, 0)),
            ],
            out_shape=[
                jax.ShapeDtypeStruct((N_NODES, D), jnp.float32),
                jax.ShapeDtypeStruct((N_NODES, D), jnp.float32),
            ],
        )(partials, h, wo, wskip, bo[None, :], wq_next)
    return pl.pallas_call(
        _combine_final_body,
        grid=grid,
        in_specs=[
            pl.BlockSpec((2, H, BN, D), lambda i: (0, 0, i, 0)),
            pl.BlockSpec((BN, D), lambda i: (i, 0)),
            full((D, D)),
            full((D, D)),
            full((1, D)),
        ],
        out_specs=pl.BlockSpec((BN, D), lambda i: (i, 0)),
        out_shape=jax.ShapeDtypeStruct((N_NODES, D), jnp.float32),
    )(partials, h, wo, wskip, bo[None, :])


def _node_init_body(x_ref, w1_ref, b1_ref, w2_ref, b2_ref, wq_ref,
                    h_ref, hq_ref):
    h = _bdot(jax.nn.relu(_bdot(x_ref[...], w1_ref[...]) + b1_ref[...]),
              w2_ref[...]) + b2_ref[...]
    h_ref[...] = h
    hq_ref[...] = _bdot(h, wq_ref[...])


def _node_init(x, p, BN=2000):
    grid = (N_NODES // BN,)
    full = lambda shape: pl.BlockSpec(shape, lambda i: (0, 0))
    return pl.pallas_call(
        _node_init_body,
        grid=grid,
        in_specs=[
            pl.BlockSpec((BN, NODE_DIM), lambda i: (i, 0)),
            full((NODE_DIM, D)),
            full((1, D)),
            full((D, D)),
            full((1, D)),
            full((D, D)),
        ],
        out_specs=[
            pl.BlockSpec((BN, D), lambda i: (i, 0)),
            pl.BlockSpec((BN, D), lambda i: (i, 0)),
        ],
        out_shape=[
            jax.ShapeDtypeStruct((N_NODES, D), jnp.float32),
            jax.ShapeDtypeStruct((N_NODES, D), jnp.float32),
        ],
    )(x, p["node_w1"], p["node_b1"][None, :], p["node_w2"],
      p["node_b2"][None, :], p["l0_wq"])


def _embeddings(src, dst, t, ef, node_h, hq0, p, zeros_acc):
    E = src.shape[0]
    BE = 2000
    eht = _edge_ht(ef, t, p, E, BE)
    h = node_h
    hq = hq0
    for l in range(N_LAYERS):
        gq, gs = _sc_gather(hq, h, src, dst, E)
        pay0, pay1 = _edge_payload(gq, gs, eht, p["l%d_wk" % l], p["l%d_wv" % l],
                                   E, BE)
        partials = _sc_scatter(pay0, pay1, dst, zeros_acc, E)[:, :, :N_NODES, :]
        if l + 1 < N_LAYERS:
            h, hq = _combine(partials, h, p["l%d_wo" % l], p["l%d_wskip" % l],
                             p["l%d_bo" % l], p["l%d_wq" % (l + 1)])
        else:
            h = _combine(partials, h, p["l%d_wo" % l], p["l%d_wskip" % l],
                         p["l%d_bo" % l], None)
    return h


def kernel(src_org_edge_feat, src_edge_to_time, src_center_node_idx,
           src_neigh_edge, src_node_features, current_time, label, params):
    p = params
    zeros_acc = jnp.zeros((_NPAD, D), jnp.float32)

    node_h, hq0 = _node_init(src_node_features, p)

    src_a = src_neigh_edge[:, 0]
    dst_a = src_neigh_edge[:, 1]
    emb = _embeddings(src_a, dst_a, src_edge_to_time, src_org_edge_feat,
                      node_h, hq0, p, zeros_acc)

    ne5 = src_neigh_edge.reshape(N_EDGES // 5, 5, 2)[:, 1:, :].reshape(-1, 2)
    t5 = src_edge_to_time.reshape(N_EDGES // 5, 5)[:, 1:].reshape(-1)
    ef5 = src_org_edge_feat.reshape(N_EDGES // 5, 5, EDGE_DIM)[:, 1:, :].reshape(-1, EDGE_DIM)
    emb_aug = _embeddings(ne5[:, 0], ne5[:, 1], t5, ef5,
                          node_h, hq0, p, zeros_acc)

    root = emb[src_center_node_idx]
    aug_root = emb_aug[src_center_node_idx]

    gh = jax.nn.relu(root @ p["gdn_w1"] + p["gdn_b1"])
    anom_score = gh @ p["gdn_w2"] + p["gdn_b2"]
    group = (current_time // 1000.0).astype(jnp.int32).reshape(-1, 1)
    ah = jax.nn.relu(root @ p["aff_w1"] + p["aff_b1"])
    logits = (ah @ p["aff_w2"] + p["aff_b2"]).reshape(-1)
    root_cat = jnp.concatenate([root, aug_root], axis=0)
    group2 = jnp.tile(group, (2, 1))
    dev2 = jnp.tile(jax.lax.stop_gradient(anom_score), (2, 1))
    return (logits, anom_score, current_time, root_cat, group2, dev2)
